# baseline (speedup 1.0000x reference)
import functools

import jax
import jax.numpy as jnp
from jax import lax
from jax.experimental import pallas as pl
from jax.experimental.pallas import tpu as pltpu
from jax.experimental.pallas import tpu_sc as plsc

_NUM_CORES = 2
_NUM_SUBCORES = 16
_NW = _NUM_CORES * _NUM_SUBCORES
_CHUNK = 128
_NRING = 5
_DEPTH = 3


def _emb_body(table_hbm, idx_hbm, out_hbm, idx_v, *scratch, cpw):
    rows = scratch[:_NRING]
    gsems = scratch[_NRING:2 * _NRING]
    wsems = scratch[2 * _NRING:3 * _NRING]

    wid = lax.axis_index("s") * _NUM_CORES + lax.axis_index("c")
    c0 = wid * cpw
    pltpu.sync_copy(idx_hbm.at[wid], idx_v)

    def start_gather(g, b):
        pltpu.async_copy(table_hbm.at[idx_v.at[g]], rows[b], gsems[b])

    def wait_gather(g, b):
        pltpu.make_async_copy(table_hbm.at[idx_v.at[g]], rows[b],
                              gsems[b]).wait()

    def start_wb(g, b):
        pltpu.async_copy(rows[b], out_hbm.at[pl.ds((c0 + g) * _CHUNK, _CHUNK)],
                         wsems[b])

    def wait_wb(g, b):
        pltpu.make_async_copy(rows[b],
                              out_hbm.at[pl.ds((c0 + g) * _CHUNK, _CHUNK)],
                              wsems[b]).wait()

    def iteration(k, b_k, b_lead, steady):
        lead = k + _DEPTH
        if steady or (_NRING <= lead < cpw):
            wait_wb(lead - _NRING, b_lead)
            start_gather(lead, b_lead)
        elif lead < cpw:
            start_gather(lead, b_lead)
        wait_gather(k, b_k)
        start_wb(k, b_k)

    for k in range(_DEPTH):
        start_gather(k, k % _NRING)

    lo = _NRING - _DEPTH
    hi = cpw - _DEPTH
    lo_t = -(-lo // _NRING)
    hi_t = hi // _NRING

    for k in range(0, lo_t * _NRING):
        iteration(k, k % _NRING, (k + _DEPTH) % _NRING, False)

    @pl.loop(0, hi_t - lo_t)
    def _(i):
        for s in range(_NRING):
            iteration((lo_t + i) * _NRING + s, s, (s + _DEPTH) % _NRING, True)

    for k in range(hi_t * _NRING, cpw):
        iteration(k, k % _NRING, (k + _DEPTH) % _NRING, False)

    for g in range(cpw - _NRING, cpw):
        wait_wb(g, g % _NRING)


def kernel(x, word_vectors):
    batch, hist = x.shape
    vocab, dim = word_vectors.shape
    total = batch * hist
    assert total % (_NW * _CHUNK) == 0
    cpw = total // _CHUNK // _NW
    assert cpw % _NRING == 0 and cpw // _NRING >= 2

    idx3d = x.T.reshape(_NW, cpw, _CHUNK).astype(jnp.int32)

    run = pl.kernel(
        functools.partial(_emb_body, cpw=cpw),
        out_type=jax.ShapeDtypeStruct((total, dim), jnp.float32),
        mesh=plsc.VectorSubcoreMesh(core_axis_name="c", subcore_axis_name="s"),
        scratch_types=(
            [pltpu.VMEM((cpw, _CHUNK), jnp.int32)]
            + [pltpu.VMEM((_CHUNK, dim), jnp.float32)] * _NRING
            + [pltpu.SemaphoreType.DMA] * (2 * _NRING)
        ),
    )
    out = run(word_vectors, idx3d)
    return out.reshape(hist, batch, dim).transpose(1, 0, 2)
